# Initial kernel scaffold; baseline (speedup 1.0000x reference)
import jax
import jax.numpy as jnp
from jax.experimental import pallas as pl


def kernel(X_in, ones):
    raise NotImplementedError("write your pallas kernel here")


# speedup vs baseline: 141.6293x; 141.6293x over previous
import jax
import jax.numpy as jnp
from jax import lax
from jax.experimental import pallas as pl
from jax.experimental.pallas import tpu as pltpu

_DEPTH = 21
_H = 512
_W = 512
_HB = 128


def _onehot_block(x_ref, o_ref):
    x = x_ref[0]
    cio = lax.broadcasted_iota(jnp.int32, (_DEPTH, _HB, _W), 0)
    o_ref[0] = (x[None, :, :] == cio).astype(jnp.float32)


def kernel(X_in, ones):
    del ones
    B = X_in.shape[0]
    x = X_in.reshape(B, _H, _W).astype(jnp.int32)
    grid = (B, _H // _HB)
    out = pl.pallas_call(
        _onehot_block,
        grid=grid,
        in_specs=[pl.BlockSpec((1, _HB, _W), lambda b, h: (b, h, 0))],
        out_specs=pl.BlockSpec((1, _DEPTH, _HB, _W), lambda b, h: (b, 0, h, 0)),
        out_shape=jax.ShapeDtypeStruct((B, _DEPTH, _H, _W), jnp.float32),
    )(x)
    return out
